# baseline (speedup 1.0000x reference)
import functools

import jax
import jax.numpy as jnp
from jax import lax
from jax.experimental import pallas as pl
from jax.experimental.pallas import tpu as pltpu
from jax.experimental.pallas import tpu_sc as plsc

_BATCH = 4096
_HIST = 50
_EMB = 256
_ROWS = _BATCH * _HIST
_NW = 32
_ROWS_PER_W = _ROWS // _NW
_CHUNK = 80
_NCHUNK = _ROWS_PER_W // _CHUNK
_NBUF = 4


def _gather_body(idx_hbm, table_hbm, out_hbm, idx_v, bufs, gsems, wsems):
    wid = lax.axis_index("s") * 2 + lax.axis_index("c")
    row_base = wid * _ROWS_PER_W

    pltpu.sync_copy(idx_hbm.at[wid], idx_v)

    def start_gather(c, s):
        pltpu.async_copy(table_hbm.at[idx_v.at[c]], bufs[s], gsems[s])

    def wait_gather(c, s):
        pltpu.make_async_copy(table_hbm.at[idx_v.at[c]], bufs[s], gsems[s]).wait()

    def out_slice(c):
        return out_hbm.at[pl.ds(row_base + c * _CHUNK, _CHUNK)]

    def start_write(c, s):
        pltpu.async_copy(bufs[s], out_slice(c), wsems[s])

    def wait_write(c, s):
        pltpu.make_async_copy(bufs[s], out_slice(c), wsems[s]).wait()

    for s in range(_NBUF):
        start_gather(s, s)

    def body(g, carry):
        c_base = g * _NBUF
        for s in range(_NBUF):
            c = c_base + s
            wait_gather(c, s)
            start_write(c, s)
        for s in range(_NBUF):
            c = c_base + s
            wait_write(c, s)

            @pl.when(c + _NBUF < _NCHUNK)
            def _():
                start_gather(c + _NBUF, s)

        return carry

    lax.fori_loop(0, _NCHUNK // _NBUF, body, 0)


@jax.jit
def _lookup(indices_3d, ent_embeds):
    mesh = plsc.VectorSubcoreMesh(core_axis_name="c", subcore_axis_name="s")
    run = functools.partial(
        pl.kernel,
        out_type=jax.ShapeDtypeStruct((_ROWS, _EMB), jnp.float32),
        mesh=mesh,
        scratch_types=[
            pltpu.VMEM((_NCHUNK, _CHUNK), jnp.int32),
            tuple(pltpu.VMEM((_CHUNK, _EMB), jnp.float32) for _ in range(_NBUF)),
            tuple(pltpu.SemaphoreType.DMA for _ in range(_NBUF)),
            tuple(pltpu.SemaphoreType.DMA for _ in range(_NBUF)),
        ],
    )(_gather_body)
    return run(indices_3d, ent_embeds)


def kernel(indices, ent_embeds):
    idx3 = indices.reshape(_NW, _NCHUNK, _CHUNK).astype(jnp.int32)
    out = _lookup(idx3, ent_embeds)
    return out.reshape(_BATCH, _HIST, _EMB)


# speedup vs baseline: 3.1121x; 3.1121x over previous
import functools

import jax
import jax.numpy as jnp
from jax import lax
from jax.experimental import pallas as pl
from jax.experimental.pallas import tpu as pltpu
from jax.experimental.pallas import tpu_sc as plsc

_BATCH = 4096
_HIST = 50
_EMB = 256
_ROWS = _BATCH * _HIST
_NW = 32
_ROWS_PER_W = _ROWS // _NW
_CHUNK = 80
_NCHUNK = _ROWS_PER_W // _CHUNK
_NBUF = 4


def _gather_body(idx_hbm, table_hbm, out_hbm, idx_v, bufs, gsems, wsems):
    wid = lax.axis_index("s") * 2 + lax.axis_index("c")
    row_base = wid * _ROWS_PER_W

    pltpu.sync_copy(idx_hbm.at[wid], idx_v)

    def start_gather(c, s):
        pltpu.async_copy(table_hbm.at[idx_v.at[c]], bufs[s], gsems[s])

    def wait_gather(c, s):
        pltpu.make_async_copy(table_hbm.at[idx_v.at[c]], bufs[s], gsems[s]).wait()

    def out_slice(c):
        return out_hbm.at[pl.ds(row_base + c * _CHUNK, _CHUNK)]

    def start_write(c, s):
        pltpu.async_copy(bufs[s], out_slice(c), wsems[s])

    def wait_write(c, s):
        pltpu.make_async_copy(bufs[s], out_slice(c), wsems[s]).wait()

    for s in range(_NBUF):
        start_gather(s, s)

    def body(g, carry):
        c_base = g * _NBUF
        for s in range(_NBUF):
            c = c_base + s
            wait_gather(c, s)
            start_write(c, s)
        for s in range(_NBUF):
            c = c_base + s
            wait_write(c, s)

            @pl.when(c + _NBUF < _NCHUNK)
            def _():
                start_gather(c + _NBUF, s)

        return carry

    lax.fori_loop(0, _NCHUNK // _NBUF, body, 0)


@jax.jit
def _lookup(indices_3d, ent_embeds):
    mesh = plsc.VectorSubcoreMesh(core_axis_name="c", subcore_axis_name="s")
    run = functools.partial(
        pl.kernel,
        out_type=jax.ShapeDtypeStruct((_ROWS, _EMB), jnp.float32),
        mesh=mesh,
        scratch_types=[
            pltpu.VMEM((_NCHUNK, _CHUNK), jnp.int32),
            tuple(pltpu.VMEM((_CHUNK, _EMB), jnp.float32) for _ in range(_NBUF)),
            tuple(pltpu.SemaphoreType.DMA for _ in range(_NBUF)),
            tuple(pltpu.SemaphoreType.DMA for _ in range(_NBUF)),
        ],
    )(_gather_body)
    return run(indices_3d, ent_embeds)


def kernel(indices, ent_embeds):
    idx3 = indices.T.reshape(_NW, _NCHUNK, _CHUNK).astype(jnp.int32)
    out = _lookup(idx3, ent_embeds)
    return out.reshape(_HIST, _BATCH, _EMB).transpose(1, 0, 2)
